# Initial kernel scaffold; baseline (speedup 1.0000x reference)
import jax
import jax.numpy as jnp
from jax.experimental import pallas as pl

E = 8
P = 1
D = 768
EPS = 1e-4
TILE = 512


def _body(x_ref, pat_ref, W_ref, b_ref, o_ref):
    x = x_ref[...]
    pats = pat_ref[...]
    logits = jax.lax.dot_general(
        x, pats, (((1,), (1,)), ((), ())),
        preferred_element_type=jnp.float32)
    s = jax.nn.sigmoid(logits)
    presence = jnp.prod(s.reshape(s.shape[0], E, P), axis=2)
    g = jnp.where(presence > EPS, presence, 0.0)
    acc = jnp.dot(g, b_ref[...], preferred_element_type=jnp.float32)
    for e in range(E):
        y = jnp.dot(x, W_ref[e], preferred_element_type=jnp.float32)
        acc = acc + g[:, e:e + 1] * y
    o_ref[...] = acc


@jax.jit
def kernel(x, patterns, W, b):
    T = x.shape[0]
    pats = patterns.reshape(E * P, D)
    grid = (T // TILE,)
    return pl.pallas_call(
        _body,
        grid=grid,
        in_specs=[
            pl.BlockSpec((TILE, D), lambda i: (i, 0)),
            pl.BlockSpec((E * P, D), lambda i: (0, 0)),
            pl.BlockSpec((E, D, D), lambda i: (0, 0, 0)),
            pl.BlockSpec((E, D), lambda i: (0, 0)),
        ],
        out_specs=pl.BlockSpec((TILE, D), lambda i: (i, 0)),
        out_shape=jax.ShapeDtypeStruct((T, D), x.dtype),
    )(x, pats, W, b)


# speedup vs baseline: 1.0212x; 1.0212x over previous
import jax
import jax.numpy as jnp
from jax.experimental import pallas as pl

E = 8
P = 1
D = 768
EPS = 1e-4
TILE = 512


def _body(x_ref, pat_ref, W_ref, b_ref, o_ref):
    x = x_ref[...]
    pats = pat_ref[...]
    logits = jax.lax.dot_general(
        x, pats, (((1,), (1,)), ((), ())),
        preferred_element_type=jnp.float32)
    s = jax.nn.sigmoid(logits)
    sr = s.reshape(s.shape[0], E, P)
    presence = sr[:, :, 0]
    for p in range(1, P):
        presence = presence * sr[:, :, p]
    g = jnp.where(presence > EPS, presence, 0.0)
    acc = jnp.dot(g, b_ref[...], preferred_element_type=jnp.float32)
    for e in range(E):
        y = jnp.dot(x, W_ref[e], preferred_element_type=jnp.float32)
        acc = acc + g[:, e:e + 1] * y
    o_ref[...] = acc


@jax.jit
def kernel(x, patterns, W, b):
    T = x.shape[0]
    pats = patterns.reshape(E * P, D)
    grid = (T // TILE,)
    return pl.pallas_call(
        _body,
        grid=grid,
        in_specs=[
            pl.BlockSpec((TILE, D), lambda i: (i, 0)),
            pl.BlockSpec((E * P, D), lambda i: (0, 0)),
            pl.BlockSpec((E, D, D), lambda i: (0, 0, 0)),
            pl.BlockSpec((E, D), lambda i: (0, 0)),
        ],
        out_specs=pl.BlockSpec((TILE, D), lambda i: (i, 0)),
        out_shape=jax.ShapeDtypeStruct((T, D), x.dtype),
    )(x, pats, W, b)
